# Initial kernel scaffold; baseline (speedup 1.0000x reference)
import jax
import jax.numpy as jnp
from jax.experimental import pallas as pl


def kernel(input_data, FRB_weights, c1, c2):
    raise NotImplementedError("write your pallas kernel here")


# speedup vs baseline: 1.5545x; 1.5545x over previous
import jax
import jax.numpy as jnp
from jax.experimental import pallas as pl

_S = 8192
_B = 512
_f32 = jnp.float32


def _sum32(a):
    acc = (a[0:8] + a[8:16]) + a[16:24]
    acc = acc + a[24:32]
    u = acc[0:4] + acc[4:8]
    v = u[0:2] + u[2:4]
    return v[0:1] + v[1:2]


def _km_block(x_ref, m_ref, s1_ref, s2_ref, c1r_ref, c1c_ref, c2r_ref,
              c2c_ref, out_ref):
    B = out_ref.shape[1]
    a1 = jnp.abs(s1_ref[...])
    a2 = jnp.abs(s2_ref[...])
    slo = jnp.minimum(a1, a2) + _f32(1e-6)
    shi = jnp.maximum(a1, a2) + _f32(1e-6)
    m = m_ref[...]

    es = []
    eb = []
    for k in range(8):
        xkb = jnp.broadcast_to(x_ref[k:k + 1, :], (32, B))
        d = xkb - jnp.broadcast_to(m[:, k:k + 1], (32, B))
        qs = d / jnp.broadcast_to(slo[:, k:k + 1], (32, B))
        qb = d / jnp.broadcast_to(shi[:, k:k + 1], (32, B))
        es.append(jnp.exp((qs * qs) * _f32(-0.5)))
        eb.append(jnp.exp((qb * qb) * _f32(-0.5)))

    def prod8(e):
        u0 = e[0] * e[4]
        u1 = e[1] * e[5]
        u2 = e[2] * e[6]
        u3 = e[3] * e[7]
        return (u0 * u2) * (u1 * u3)

    LL = prod8(es)
    UU = prod8(eb)

    subi = jax.lax.broadcasted_iota(jnp.int32, (32, 32), 0)
    lanei = jax.lax.broadcasted_iota(jnp.int32, (32, 32), 1)

    def onehot_perm(crow_ref, ccol_ref):
        crow = jnp.broadcast_to(crow_ref[...], (32, 32))
        ccol = jnp.broadcast_to(ccol_ref[...], (32, 32))
        lt = ccol < crow
        eq = ccol == crow
        cnt = jnp.where(lt | (eq & (subi < lanei)), 1, 0)
        rank_row = jnp.sum(cnt, axis=0, keepdims=True)
        P = jnp.broadcast_to(rank_row, (32, 32)) == subi
        return P

    P1 = onehot_perm(c1r_ref, c1c_ref)
    P2 = onehot_perm(c2r_ref, c2c_ref)

    def sorted_vals(P, crow_ref):
        cb = jnp.broadcast_to(crow_ref[...], (32, 32))
        return jnp.sum(jnp.where(P, cb, _f32(0)), axis=1, keepdims=True)

    c1s = sorted_vals(P1, c1r_ref)
    c2s = sorted_vals(P2, c2r_ref)

    def permute_rows(P, a):
        pf = P.astype(_f32)
        out = jnp.zeros((32, B), _f32)
        for l in range(32):
            out = out + (jnp.broadcast_to(pf[:, l:l + 1], (32, B)) *
                         jnp.broadcast_to(a[l:l + 1, :], (32, B)))
        return out

    L_UU = permute_rows(P1, UU)
    L_LL = permute_rows(P1, LL)
    R_UU = permute_rows(P2, UU)
    R_LL = permute_rows(P2, LL)

    c1s_b = jnp.broadcast_to(c1s, (32, B))
    c2s_b = jnp.broadcast_to(c2s, (32, B))
    s0 = _sum32(c1s_b * L_LL)
    t0 = _sum32(L_LL)
    s0r = _sum32(c2s_b * R_UU)
    t0r = _sum32(R_UU)
    dL = L_UU - L_LL
    dR = R_LL - R_UU
    ndL = c1s_b * dL
    ndR = c2s_b * dR
    q = s0 / t0
    qr = s0r / t0r

    cn = jnp.zeros((1, B), _f32)
    cd = jnp.zeros((1, B), _f32)
    cmn = jnp.full((1, B), jnp.inf, _f32)
    bestv = jnp.full((1, B), jnp.inf, _f32)
    besti = jnp.zeros((1, B), jnp.int32)
    for t in range(32):
        cn = cn + ndL[t:t + 1]
        cd = cd + dL[t:t + 1]
        ratio = (s0 + cn) / (t0 + cd)
        cmn = jnp.minimum(cmn, ratio)
        lout = jnp.minimum(cmn, q)
        keepv = (bestv < lout) | (bestv != bestv)
        keepi = keepv | (bestv == lout)
        bestv = jnp.where(keepv, bestv, lout)
        besti = jnp.where(keepi, besti, jnp.full((1, B), t, jnp.int32))
    L_loc = besti

    cn = jnp.zeros((1, B), _f32)
    cd = jnp.zeros((1, B), _f32)
    cmx = jnp.full((1, B), -jnp.inf, _f32)
    bestv = jnp.full((1, B), -jnp.inf, _f32)
    besti = jnp.zeros((1, B), jnp.int32)
    for t in range(32):
        cn = cn + ndR[t:t + 1]
        cd = cd + dR[t:t + 1]
        ratio = (s0r + cn) / (t0r + cd)
        cmx = jnp.maximum(cmx, ratio)
        rout = jnp.maximum(cmx, qr)
        keepv = (bestv > rout) | (bestv != bestv)
        keepi = keepv | (bestv == rout)
        bestv = jnp.where(keepv, bestv, rout)
        besti = jnp.where(keepi, besti, jnp.full((1, B), t, jnp.int32))
    R_loc = besti

    rowi = jax.lax.broadcasted_iota(jnp.int32, (32, B), 0)
    selL = jnp.where(rowi <= jnp.broadcast_to(L_loc, (32, B)), L_UU, L_LL)
    selR = jnp.where(rowi <= jnp.broadcast_to(R_loc, (32, B)), R_LL, R_UU)
    c1n_b = jnp.broadcast_to(c1c_ref[...], (32, B))
    c2n_b = jnp.broadcast_to(c2c_ref[...], (32, B))
    out_left = _sum32(c1n_b * selL) / _sum32(selL)
    out_right = _sum32(c2n_b * selR) / _sum32(selR)
    out_ref[...] = (out_right + out_left) / _f32(2.0)


def kernel(input_data, FRB_weights, c1, c2):
    xT = input_data.T
    m = FRB_weights[0:256].reshape(32, 8)
    s1 = FRB_weights[1:257].reshape(32, 8)
    s2 = FRB_weights[2:258].reshape(32, 8)
    c1r = c1.reshape(1, 32)
    c1c = c1.reshape(32, 1)
    c2r = c2.reshape(1, 32)
    c2c = c2.reshape(32, 1)
    rep = pl.BlockSpec((32, 8), lambda i: (0, 0))
    out = pl.pallas_call(
        _km_block,
        grid=(_S // _B,),
        in_specs=[
            pl.BlockSpec((8, _B), lambda i: (0, i)),
            rep, rep, rep,
            pl.BlockSpec((1, 32), lambda i: (0, 0)),
            pl.BlockSpec((32, 1), lambda i: (0, 0)),
            pl.BlockSpec((1, 32), lambda i: (0, 0)),
            pl.BlockSpec((32, 1), lambda i: (0, 0)),
        ],
        out_specs=pl.BlockSpec((1, _B), lambda i: (0, i)),
        out_shape=jax.ShapeDtypeStruct((1, _S), jnp.float32),
    )(xT, m, s1, s2, c1r, c1c, c2r, c2c)
    return out.reshape(_S)


# speedup vs baseline: 2.3166x; 1.4902x over previous
import jax
import jax.numpy as jnp
from jax.experimental import pallas as pl

_S = 8192
_B = 512
_f32 = jnp.float32


def _sum32(a):
    acc = (a[0:8] + a[8:16]) + a[16:24]
    acc = acc + a[24:32]
    u = acc[0:4] + acc[4:8]
    v = u[0:2] + u[2:4]
    return v[0:1] + v[1:2]


def _km_block(x_ref, m_ref, s1_ref, s2_ref, c1r_ref, c1c_ref, c2r_ref,
              c2c_ref, out_ref):
    B = out_ref.shape[1]
    a1 = jnp.abs(s1_ref[...])
    a2 = jnp.abs(s2_ref[...])
    slo = jnp.minimum(a1, a2) + _f32(1e-6)
    shi = jnp.maximum(a1, a2) + _f32(1e-6)
    m = m_ref[...]

    es = []
    eb = []
    for k in range(8):
        xkb = jnp.broadcast_to(x_ref[k:k + 1, :], (32, B))
        d = xkb - jnp.broadcast_to(m[:, k:k + 1], (32, B))
        qs = d / jnp.broadcast_to(slo[:, k:k + 1], (32, B))
        qb = d / jnp.broadcast_to(shi[:, k:k + 1], (32, B))
        es.append(jnp.exp((qs * qs) * _f32(-0.5)))
        eb.append(jnp.exp((qb * qb) * _f32(-0.5)))

    def prod8(e):
        u0 = e[0] * e[4]
        u1 = e[1] * e[5]
        u2 = e[2] * e[6]
        u3 = e[3] * e[7]
        return (u0 * u2) * (u1 * u3)

    LL = prod8(es)
    UU = prod8(eb)

    subi = jax.lax.broadcasted_iota(jnp.int32, (32, 32), 0)
    lanei = jax.lax.broadcasted_iota(jnp.int32, (32, 32), 1)

    def onehot_perm(crow_ref, ccol_ref):
        crow = jnp.broadcast_to(crow_ref[...], (32, 32))
        ccol = jnp.broadcast_to(ccol_ref[...], (32, 32))
        lt = ccol < crow
        eq = ccol == crow
        cnt = jnp.where(lt | (eq & (subi < lanei)), 1, 0)
        rank_row = jnp.sum(cnt, axis=0, keepdims=True)
        P = jnp.broadcast_to(rank_row, (32, 32)) == subi
        return P

    P1 = onehot_perm(c1r_ref, c1c_ref)
    P2 = onehot_perm(c2r_ref, c2c_ref)

    def sorted_vals(P, crow_ref):
        cb = jnp.broadcast_to(crow_ref[...], (32, 32))
        return jnp.sum(jnp.where(P, cb, _f32(0)), axis=1, keepdims=True)

    c1s = sorted_vals(P1, c1r_ref)
    c2s = sorted_vals(P2, c2r_ref)

    def permute_rows(P, a):
        return jax.lax.dot_general(P.astype(_f32), a, (((1,), (0,)), ((), ())),
                                   preferred_element_type=_f32)

    L_UU = permute_rows(P1, UU)
    L_LL = permute_rows(P1, LL)
    R_UU = permute_rows(P2, UU)
    R_LL = permute_rows(P2, LL)

    c1s_b = jnp.broadcast_to(c1s, (32, B))
    c2s_b = jnp.broadcast_to(c2s, (32, B))
    s0 = _sum32(c1s_b * L_LL)
    t0 = _sum32(L_LL)
    s0r = _sum32(c2s_b * R_UU)
    t0r = _sum32(R_UU)
    dL = L_UU - L_LL
    dR = R_LL - R_UU
    ndL = c1s_b * dL
    ndR = c2s_b * dR
    q = s0 / t0
    qr = s0r / t0r

    cn = jnp.zeros((1, B), _f32)
    cd = jnp.zeros((1, B), _f32)
    cmn = jnp.full((1, B), jnp.inf, _f32)
    bestv = jnp.full((1, B), jnp.inf, _f32)
    besti = jnp.zeros((1, B), jnp.int32)
    for t in range(32):
        cn = cn + ndL[t:t + 1]
        cd = cd + dL[t:t + 1]
        ratio = (s0 + cn) / (t0 + cd)
        cmn = jnp.minimum(cmn, ratio)
        lout = jnp.minimum(cmn, q)
        keepv = (bestv < lout) | (bestv != bestv)
        keepi = keepv | (bestv == lout)
        bestv = jnp.where(keepv, bestv, lout)
        besti = jnp.where(keepi, besti, jnp.full((1, B), t, jnp.int32))
    L_loc = besti

    cn = jnp.zeros((1, B), _f32)
    cd = jnp.zeros((1, B), _f32)
    cmx = jnp.full((1, B), -jnp.inf, _f32)
    bestv = jnp.full((1, B), -jnp.inf, _f32)
    besti = jnp.zeros((1, B), jnp.int32)
    for t in range(32):
        cn = cn + ndR[t:t + 1]
        cd = cd + dR[t:t + 1]
        ratio = (s0r + cn) / (t0r + cd)
        cmx = jnp.maximum(cmx, ratio)
        rout = jnp.maximum(cmx, qr)
        keepv = (bestv > rout) | (bestv != bestv)
        keepi = keepv | (bestv == rout)
        bestv = jnp.where(keepv, bestv, rout)
        besti = jnp.where(keepi, besti, jnp.full((1, B), t, jnp.int32))
    R_loc = besti

    rowi = jax.lax.broadcasted_iota(jnp.int32, (32, B), 0)
    selL = jnp.where(rowi <= jnp.broadcast_to(L_loc, (32, B)), L_UU, L_LL)
    selR = jnp.where(rowi <= jnp.broadcast_to(R_loc, (32, B)), R_LL, R_UU)
    c1n_b = jnp.broadcast_to(c1c_ref[...], (32, B))
    c2n_b = jnp.broadcast_to(c2c_ref[...], (32, B))
    out_left = _sum32(c1n_b * selL) / _sum32(selL)
    out_right = _sum32(c2n_b * selR) / _sum32(selR)
    out_ref[...] = (out_right + out_left) / _f32(2.0)


def kernel(input_data, FRB_weights, c1, c2):
    xT = input_data.T
    m = FRB_weights[0:256].reshape(32, 8)
    s1 = FRB_weights[1:257].reshape(32, 8)
    s2 = FRB_weights[2:258].reshape(32, 8)
    c1r = c1.reshape(1, 32)
    c1c = c1.reshape(32, 1)
    c2r = c2.reshape(1, 32)
    c2c = c2.reshape(32, 1)
    rep = pl.BlockSpec((32, 8), lambda i: (0, 0))
    out = pl.pallas_call(
        _km_block,
        grid=(_S // _B,),
        in_specs=[
            pl.BlockSpec((8, _B), lambda i: (0, i)),
            rep, rep, rep,
            pl.BlockSpec((1, 32), lambda i: (0, 0)),
            pl.BlockSpec((32, 1), lambda i: (0, 0)),
            pl.BlockSpec((1, 32), lambda i: (0, 0)),
            pl.BlockSpec((32, 1), lambda i: (0, 0)),
        ],
        out_specs=pl.BlockSpec((1, _B), lambda i: (0, i)),
        out_shape=jax.ShapeDtypeStruct((1, _S), jnp.float32),
    )(xT, m, s1, s2, c1r, c1c, c2r, c2c)
    return out.reshape(_S)


# speedup vs baseline: 2.6184x; 1.1303x over previous
import jax
import jax.numpy as jnp
from jax.experimental import pallas as pl

_S = 8192
_B = 1024
_f32 = jnp.float32


def _sum32(a):
    acc = (a[0:8] + a[8:16]) + a[16:24]
    acc = acc + a[24:32]
    u = acc[0:4] + acc[4:8]
    v = u[0:2] + u[2:4]
    return v[0:1] + v[1:2]


def _km_block(x_ref, m_ref, s1_ref, s2_ref, c1r_ref, c1c_ref, c2r_ref,
              c2c_ref, out_ref):
    B = out_ref.shape[1]
    a1 = jnp.abs(s1_ref[...])
    a2 = jnp.abs(s2_ref[...])
    slo = jnp.minimum(a1, a2) + _f32(1e-6)
    shi = jnp.maximum(a1, a2) + _f32(1e-6)
    m = m_ref[...]

    es = []
    eb = []
    for k in range(8):
        xkb = jnp.broadcast_to(x_ref[k:k + 1, :], (32, B))
        d = xkb - jnp.broadcast_to(m[:, k:k + 1], (32, B))
        qs = d / jnp.broadcast_to(slo[:, k:k + 1], (32, B))
        qb = d / jnp.broadcast_to(shi[:, k:k + 1], (32, B))
        es.append(jnp.exp((qs * qs) * _f32(-0.5)))
        eb.append(jnp.exp((qb * qb) * _f32(-0.5)))

    def prod8(e):
        u0 = e[0] * e[4]
        u1 = e[1] * e[5]
        u2 = e[2] * e[6]
        u3 = e[3] * e[7]
        return (u0 * u2) * (u1 * u3)

    LL = prod8(es)
    UU = prod8(eb)

    subi = jax.lax.broadcasted_iota(jnp.int32, (32, 32), 0)
    lanei = jax.lax.broadcasted_iota(jnp.int32, (32, 32), 1)

    def onehot_perm(crow_ref, ccol_ref):
        crow = jnp.broadcast_to(crow_ref[...], (32, 32))
        ccol = jnp.broadcast_to(ccol_ref[...], (32, 32))
        lt = ccol < crow
        eq = ccol == crow
        cnt = jnp.where(lt | (eq & (subi < lanei)), 1, 0)
        rank_row = jnp.sum(cnt, axis=0, keepdims=True)
        P = jnp.broadcast_to(rank_row, (32, 32)) == subi
        return P

    P1 = onehot_perm(c1r_ref, c1c_ref)
    P2 = onehot_perm(c2r_ref, c2c_ref)

    def sorted_vals(P, crow_ref):
        cb = jnp.broadcast_to(crow_ref[...], (32, 32))
        return jnp.sum(jnp.where(P, cb, _f32(0)), axis=1, keepdims=True)

    c1s = sorted_vals(P1, c1r_ref)
    c2s = sorted_vals(P2, c2r_ref)

    def permute_rows(P, a):
        return jax.lax.dot_general(P.astype(_f32), a, (((1,), (0,)), ((), ())),
                                   preferred_element_type=_f32)

    L_UU = permute_rows(P1, UU)
    L_LL = permute_rows(P1, LL)
    R_UU = permute_rows(P2, UU)
    R_LL = permute_rows(P2, LL)

    c1s_b = jnp.broadcast_to(c1s, (32, B))
    c2s_b = jnp.broadcast_to(c2s, (32, B))
    s0 = _sum32(c1s_b * L_LL)
    t0 = _sum32(L_LL)
    s0r = _sum32(c2s_b * R_UU)
    t0r = _sum32(R_UU)
    dL = L_UU - L_LL
    dR = R_LL - R_UU
    ndL = c1s_b * dL
    ndR = c2s_b * dR
    q = s0 / t0
    qr = s0r / t0r

    cn = jnp.zeros((1, B), _f32)
    cd = jnp.zeros((1, B), _f32)
    cmn = jnp.full((1, B), jnp.inf, _f32)
    bestv = jnp.full((1, B), jnp.inf, _f32)
    besti = jnp.zeros((1, B), jnp.int32)
    for t in range(32):
        cn = cn + ndL[t:t + 1]
        cd = cd + dL[t:t + 1]
        ratio = (s0 + cn) / (t0 + cd)
        cmn = jnp.minimum(cmn, ratio)
        lout = jnp.minimum(cmn, q)
        keepv = (bestv < lout) | (bestv != bestv)
        keepi = keepv | (bestv == lout)
        bestv = jnp.where(keepv, bestv, lout)
        besti = jnp.where(keepi, besti, jnp.full((1, B), t, jnp.int32))
    L_loc = besti

    cn = jnp.zeros((1, B), _f32)
    cd = jnp.zeros((1, B), _f32)
    cmx = jnp.full((1, B), -jnp.inf, _f32)
    bestv = jnp.full((1, B), -jnp.inf, _f32)
    besti = jnp.zeros((1, B), jnp.int32)
    for t in range(32):
        cn = cn + ndR[t:t + 1]
        cd = cd + dR[t:t + 1]
        ratio = (s0r + cn) / (t0r + cd)
        cmx = jnp.maximum(cmx, ratio)
        rout = jnp.maximum(cmx, qr)
        keepv = (bestv > rout) | (bestv != bestv)
        keepi = keepv | (bestv == rout)
        bestv = jnp.where(keepv, bestv, rout)
        besti = jnp.where(keepi, besti, jnp.full((1, B), t, jnp.int32))
    R_loc = besti

    rowi = jax.lax.broadcasted_iota(jnp.int32, (32, B), 0)
    selL = jnp.where(rowi <= jnp.broadcast_to(L_loc, (32, B)), L_UU, L_LL)
    selR = jnp.where(rowi <= jnp.broadcast_to(R_loc, (32, B)), R_LL, R_UU)
    c1n_b = jnp.broadcast_to(c1c_ref[...], (32, B))
    c2n_b = jnp.broadcast_to(c2c_ref[...], (32, B))
    out_left = _sum32(c1n_b * selL) / _sum32(selL)
    out_right = _sum32(c2n_b * selR) / _sum32(selR)
    out_ref[...] = (out_right + out_left) / _f32(2.0)


def kernel(input_data, FRB_weights, c1, c2):
    xT = input_data.T
    m = FRB_weights[0:256].reshape(32, 8)
    s1 = FRB_weights[1:257].reshape(32, 8)
    s2 = FRB_weights[2:258].reshape(32, 8)
    c1r = c1.reshape(1, 32)
    c1c = c1.reshape(32, 1)
    c2r = c2.reshape(1, 32)
    c2c = c2.reshape(32, 1)
    rep = pl.BlockSpec((32, 8), lambda i: (0, 0))
    out = pl.pallas_call(
        _km_block,
        grid=(_S // _B,),
        in_specs=[
            pl.BlockSpec((8, _B), lambda i: (0, i)),
            rep, rep, rep,
            pl.BlockSpec((1, 32), lambda i: (0, 0)),
            pl.BlockSpec((32, 1), lambda i: (0, 0)),
            pl.BlockSpec((1, 32), lambda i: (0, 0)),
            pl.BlockSpec((32, 1), lambda i: (0, 0)),
        ],
        out_specs=pl.BlockSpec((1, _B), lambda i: (0, i)),
        out_shape=jax.ShapeDtypeStruct((1, _S), jnp.float32),
    )(xT, m, s1, s2, c1r, c1c, c2r, c2c)
    return out.reshape(_S)
